# baseline (speedup 1.0000x reference)
import functools

import jax
import jax.numpy as jnp
from jax.experimental import pallas as pl
from jax.experimental.pallas import tpu as pltpu


def _body(x_ref, adj_a_ref, adj_b_ref, w1_ref, b1_ref, g_ref, bb_ref,
          w2_ref, b2_ref, w3_ref, b3_ref, out_ref, h1_s, t_s, *, m, bm, eps):
    s = pl.program_id(0)
    hb = bm // 2

    @pl.when(s == 0)
    def _phase_h1():
        h1_s[...] = (
            jnp.dot(x_ref[...], w1_ref[...], preferred_element_type=jnp.float32)
            + b1_ref[...]
        )

    @pl.when((s >= 1) & (s <= m))
    def _phase_a():
        for half, ref in ((0, adj_a_ref), (1, adj_b_ref)):
            row0 = (s - 1) * bm + half * hb
            adj = ref[0]
            agg = jnp.dot(adj, h1_s[...], preferred_element_type=jnp.float32)
            hself = h1_s[pl.ds(row0, hb), :]
            cat = jnp.concatenate([hself, agg], axis=1)
            mu = jnp.mean(cat, axis=-1, keepdims=True)
            var = jnp.mean(jnp.square(cat - mu), axis=-1, keepdims=True)
            ln = (cat - mu) * jax.lax.rsqrt(var + eps) * g_ref[...] + bb_ref[...]
            h = jnp.maximum(ln, 0.0)
            t_s[pl.ds(row0, hb), :] = (
                jnp.dot(h, w2_ref[...], preferred_element_type=jnp.float32)
                + b2_ref[...]
            )

    @pl.when(s >= m + 1)
    def _phase_b():
        for half, ref in ((0, adj_a_ref), (1, adj_b_ref)):
            adj = ref[0]
            h2 = jnp.maximum(
                jnp.dot(adj, t_s[...], preferred_element_type=jnp.float32), 0.0
            )
            out_ref[pl.ds(half * hb, hb), :] = (
                jnp.dot(h2, w3_ref[...], preferred_element_type=jnp.float32)
                + b3_ref[...]
            )


def kernel(x, adjs, W1, b1, ln_g, ln_b, W2, b2, W3, b3):
    n, nfeat = x.shape
    nhid = W1.shape[1]
    ncls = W3.shape[1]

    bm = 400
    assert n % bm == 0
    m = n // bm
    grid = 2 * m + 1
    hb = bm // 2
    nhalf = n // hb

    b1_2 = b1.reshape(1, nhid)
    g_2 = ln_g.reshape(1, 2 * nhid)
    bb_2 = ln_b.reshape(1, 2 * nhid)
    b2_2 = b2.reshape(1, nhid)
    b3_2 = b3.reshape(1, ncls)

    def adj_index(half):
        def f(s):
            p = jnp.where(s > m, 1, 0)
            i = jnp.where(s > m, s - (m + 1), jnp.maximum(s - 1, 0))
            return (p, 2 * i + half, 0)
        return f

    def out_index(s):
        return (jnp.where(s > m, s - (m + 1), 0), 0)

    const = lambda s: (0, 0)

    body = functools.partial(_body, m=m, bm=bm, eps=1e-5)

    return pl.pallas_call(
        body,
        grid=(grid,),
        in_specs=[
            pl.BlockSpec((n, nfeat), const),
            pl.BlockSpec((1, hb, n), adj_index(0)),
            pl.BlockSpec((1, hb, n), adj_index(1)),
            pl.BlockSpec((nfeat, nhid), const),
            pl.BlockSpec((1, nhid), const),
            pl.BlockSpec((1, 2 * nhid), const),
            pl.BlockSpec((1, 2 * nhid), const),
            pl.BlockSpec((2 * nhid, nhid), const),
            pl.BlockSpec((1, nhid), const),
            pl.BlockSpec((nhid, ncls), const),
            pl.BlockSpec((1, ncls), const),
        ],
        out_specs=pl.BlockSpec((bm, ncls), out_index),
        out_shape=jax.ShapeDtypeStruct((n, ncls), jnp.float32),
        scratch_shapes=[
            pltpu.VMEM((n, nhid), jnp.float32),
            pltpu.VMEM((n, nhid), jnp.float32),
        ],
    )(x, adjs, adjs, W1, b1_2, g_2, bb_2, W2, b2_2, W3, b3_2)


# speedup vs baseline: 1.1023x; 1.1023x over previous
import functools

import jax
import jax.numpy as jnp
from jax.experimental import pallas as pl
from jax.experimental.pallas import tpu as pltpu


def _body(x_ref, adj_ref, w1_ref, b1_ref, g_ref, bb_ref, w2_ref, b2_ref,
          w3_ref, b3_ref, out_ref, h1_s, t_s, *, m, bm, eps):
    s = pl.program_id(0)

    @pl.when(s == 0)
    def _phase_h1():
        h1_s[...] = (
            jnp.dot(x_ref[...], w1_ref[...], preferred_element_type=jnp.float32)
            + b1_ref[...]
        )

    @pl.when(s < m)
    def _phase_a():
        row0 = s * bm
        adj = adj_ref[0]
        agg = jnp.dot(adj, h1_s[...], preferred_element_type=jnp.float32)
        hself = h1_s[pl.ds(row0, bm), :]
        cat = jnp.concatenate([hself, agg], axis=1)
        mu = jnp.mean(cat, axis=-1, keepdims=True)
        var = jnp.mean(jnp.square(cat - mu), axis=-1, keepdims=True)
        ln = (cat - mu) * jax.lax.rsqrt(var + eps) * g_ref[...] + bb_ref[...]
        h = jnp.maximum(ln, 0.0)
        t_s[pl.ds(row0, bm), :] = (
            jnp.dot(h, w2_ref[...], preferred_element_type=jnp.float32)
            + b2_ref[...]
        )

    @pl.when(s >= m)
    def _phase_b():
        adj = adj_ref[0]
        h2 = jnp.maximum(
            jnp.dot(adj, t_s[...], preferred_element_type=jnp.float32), 0.0
        )
        out_ref[...] = (
            jnp.dot(h2, w3_ref[...], preferred_element_type=jnp.float32)
            + b3_ref[...]
        )


def kernel(x, adjs, W1, b1, ln_g, ln_b, W2, b2, W3, b3):
    n, nfeat = x.shape
    nhid = W1.shape[1]
    ncls = W3.shape[1]

    bm = 400
    assert n % bm == 0
    m = n // bm
    grid = 2 * m

    b1_2 = b1.reshape(1, nhid)
    g_2 = ln_g.reshape(1, 2 * nhid)
    bb_2 = ln_b.reshape(1, 2 * nhid)
    b2_2 = b2.reshape(1, nhid)
    b3_2 = b3.reshape(1, ncls)

    def adj_index(s):
        p = jnp.where(s >= m, 1, 0)
        i = jnp.where(s >= m, s - m, s)
        return (p, i, 0)

    def out_index(s):
        return (jnp.where(s >= m, s - m, 0), 0)

    const = lambda s: (0, 0)

    body = functools.partial(_body, m=m, bm=bm, eps=1e-5)

    return pl.pallas_call(
        body,
        grid=(grid,),
        in_specs=[
            pl.BlockSpec((n, nfeat), const),
            pl.BlockSpec((1, bm, n), adj_index),
            pl.BlockSpec((nfeat, nhid), const),
            pl.BlockSpec((1, nhid), const),
            pl.BlockSpec((1, 2 * nhid), const),
            pl.BlockSpec((1, 2 * nhid), const),
            pl.BlockSpec((2 * nhid, nhid), const),
            pl.BlockSpec((1, nhid), const),
            pl.BlockSpec((nhid, ncls), const),
            pl.BlockSpec((1, ncls), const),
        ],
        out_specs=pl.BlockSpec((bm, ncls), out_index),
        out_shape=jax.ShapeDtypeStruct((n, ncls), jnp.float32),
        scratch_shapes=[
            pltpu.VMEM((n, nhid), jnp.float32),
            pltpu.VMEM((n, nhid), jnp.float32),
        ],
    )(x, adjs, W1, b1_2, g_2, bb_2, W2, b2_2, W3, b3_2)
